# baseline (speedup 1.0000x reference)
import functools

import jax
import jax.numpy as jnp
from jax import lax
from jax.experimental import pallas as pl
from jax.experimental.pallas import tpu as pltpu
from jax.experimental.pallas import tpu_sc as plsc

N = 10000
E = 640000
F = 128
OUT = 64
B = 128
EPS = 1e-5

NC = 2
NS = 16
NP = 10240
RPT = NP // NS
NPA = NP + 8

C = 40
CH = 500
NB = 5
PH = 5
PCH = CH // PH


def _sc_aggregate_body(h_hbm, src_hbm, dst_hbm, out_hbm, src_v, dst_v,
                       acc, rows, gsem, ssem):
    cid = lax.axis_index("c")
    sid = lax.axis_index("s")
    sv = src_hbm.at[cid, sid]
    dv = dst_hbm.at[cid, sid]

    pltpu.sync_copy(h_hbm.at[pl.ds(sid * RPT, RPT)], acc.at[pl.ds(sid * RPT, RPT)])
    plsc.subcore_barrier()

    def gather(j, b):
        pltpu.async_copy(h_hbm.at[src_v.at[j]], rows[b], gsem[b])

    def gather_wait(j, b):
        pltpu.make_async_copy(h_hbm.at[src_v.at[j]], rows[b], gsem[b]).wait()

    def scatter(j, b):
        pltpu.async_copy(rows[b], acc.at[dst_v.at[j]], ssem[b], add=True)

    def scatter_wait(j, b):
        pltpu.make_async_copy(rows[b], acc.at[dst_v.at[j]], ssem[b]).wait()

    @pl.loop(0, PH)
    def _(ph):
        pltpu.sync_copy(sv.at[pl.ds(ph * PCH, PCH)], src_v)
        pltpu.sync_copy(dv.at[pl.ds(ph * PCH, PCH)], dst_v)

        for b in range(NB):
            gather(b, b)

        @pl.loop(0, PCH // NB - 1)
        def _(jj):
            j = jj * NB
            for b in range(NB):
                gather_wait(j + b, b)
                scatter(j + b, b)
                scatter_wait(j + b, b)
                gather(j + b + NB, b)

        jl = PCH - NB
        for b in range(NB):
            gather_wait(jl + b, b)
            scatter(jl + b, b)
        for b in range(NB):
            scatter_wait(jl + b, b)

    plsc.subcore_barrier()
    pltpu.sync_copy(acc.at[pl.ds(sid * RPT, RPT)],
                    out_hbm.at[cid].at[pl.ds(sid * RPT, RPT)])


@functools.cache
def _sc_aggregate():
    mesh = plsc.VectorSubcoreMesh(core_axis_name="c", subcore_axis_name="s",
                                  num_cores=NC, num_subcores=NS)
    return pl.kernel(
        _sc_aggregate_body,
        out_type=jax.ShapeDtypeStruct((NC, NP, F), jnp.float32),
        mesh=mesh,
        scratch_types=[
            pltpu.VMEM((PCH, C), jnp.int32),
            pltpu.VMEM((PCH, C), jnp.int32),
            pltpu.VMEM_SHARED((NPA, F), jnp.float32),
            [pltpu.VMEM((C, F), jnp.float32) for _ in range(NB)],
            [pltpu.SemaphoreType.DMA for _ in range(NB)],
            [pltpu.SemaphoreType.DMA for _ in range(NB)],
        ],
        compiler_params=pltpu.CompilerParams(use_tc_tiling_on_sc=False),
    )


def _row_mask():
    rows = lax.broadcasted_iota(jnp.int32, (NP, 1), 0)
    return (rows < N).astype(jnp.float32)


def _mlp_block(t, W1, b1, g, be, W2, b2):
    rmask = _row_mask()
    y = jnp.dot(t, W1, preferred_element_type=jnp.float32) + b1
    mu = jnp.sum(y * rmask, axis=0, keepdims=True) * (1.0 / N)
    d = y - mu
    var = jnp.sum(d * d * rmask, axis=0, keepdims=True) * (1.0 / N)
    z = g * d * lax.rsqrt(var + EPS) + be
    z = jnp.maximum(z, 0.0)
    h2 = jnp.dot(z, W2, preferred_element_type=jnp.float32) + b2
    return jnp.maximum(h2, 0.0) * rmask


def _mlp_body(p_ref, h_ref, W1_ref, b1_ref, g_ref, be_ref, W2_ref, b2_ref,
              o_ref):
    t = p_ref[0] + p_ref[1] - h_ref[...]
    o_ref[...] = _mlp_block(t, W1_ref[...], b1_ref[...], g_ref[...],
                            be_ref[...], W2_ref[...], b2_ref[...])


_tc_mlp = pl.pallas_call(
    _mlp_body,
    out_shape=jax.ShapeDtypeStruct((NP, F), jnp.float32),
)


def _final_body(p_ref, h_ref, batch_ref, W1_ref, b1_ref, g_ref, be_ref,
                W2_ref, b2_ref, Wm1_ref, bm1_ref, Wm2_ref, bm2_ref, o_ref):
    t = p_ref[0] + p_ref[1] - h_ref[...]
    hh = _mlp_block(t, W1_ref[...], b1_ref[...], g_ref[...], be_ref[...],
                    W2_ref[...], b2_ref[...])
    cols = lax.broadcasted_iota(jnp.int32, (B, NP), 0)
    mask = (cols == batch_ref[...]).astype(jnp.float32)
    pooled = jnp.dot(mask, hh, preferred_element_type=jnp.float32)
    q = jnp.dot(pooled, Wm1_ref[...], preferred_element_type=jnp.float32) + bm1_ref[...]
    q = jnp.maximum(q, 0.0)
    o_ref[...] = jnp.dot(q, Wm2_ref[...], preferred_element_type=jnp.float32) + bm2_ref[...]


_tc_final = pl.pallas_call(
    _final_body,
    out_shape=jax.ShapeDtypeStruct((B, OUT), jnp.float32),
)


def kernel(x, edge_index, batch, batch_size,
           W1_0, b1_0, g_0, be_0, W2_0, b2_0,
           W1_1, b1_1, g_1, be_1, W2_1, b2_1,
           W1_2, b1_2, g_2, be_2, W2_2, b2_2,
           Wm1, bm1, Wm2, bm2):
    src = edge_index[0].reshape(NC, NS, CH, C)
    dst = edge_index[1].reshape(NC, NS, CH, C)
    batch2d = jnp.pad(batch, (0, NP - N), constant_values=B).reshape(1, NP)
    h = jnp.pad(x, ((0, NP - N), (0, 0)))

    def row(v):
        return v.reshape(1, -1)

    layers = [
        (W1_0, b1_0, g_0, be_0, W2_0, b2_0),
        (W1_1, b1_1, g_1, be_1, W2_1, b2_1),
    ]
    agg = _sc_aggregate()
    for (W1, b1, g, be, W2, b2) in layers:
        p = agg(h, src, dst)
        h = _tc_mlp(p, h, W1, row(b1), row(g), row(be), W2, row(b2))
    p = agg(h, src, dst)
    return _tc_final(p, h, batch2d, W1_2, row(b1_2), row(g_2), row(be_2),
                     W2_2, row(b2_2), Wm1, row(bm1), Wm2, row(bm2))


# speedup vs baseline: 1.0392x; 1.0392x over previous
import functools

import jax
import jax.numpy as jnp
from jax import lax
from jax.experimental import pallas as pl
from jax.experimental.pallas import tpu as pltpu
from jax.experimental.pallas import tpu_sc as plsc

N = 10000
E = 640000
F = 128
OUT = 64
B = 128
EPS = 1e-5

NC = 2
NS = 16
NP = 10240
RPT = NP // NS
NPA = NP + 8

C = 40
CH = 500
NB = 5
PH = 2
PCH = CH // PH


def _sc_aggregate_body(h_hbm, src_hbm, dst_hbm, out_hbm, src_v, dst_v,
                       acc, rows, gsem, ssem):
    cid = lax.axis_index("c")
    sid = lax.axis_index("s")
    sv = src_hbm.at[cid, sid]
    dv = dst_hbm.at[cid, sid]

    pltpu.sync_copy(h_hbm.at[pl.ds(sid * RPT, RPT)], acc.at[pl.ds(sid * RPT, RPT)])
    plsc.subcore_barrier()

    def gather(j, b):
        pltpu.async_copy(h_hbm.at[src_v.at[j]], rows[b], gsem[b])

    def gather_wait(j, b):
        pltpu.make_async_copy(h_hbm.at[src_v.at[j]], rows[b], gsem[b]).wait()

    def scatter(j, b):
        pltpu.async_copy(rows[b], acc.at[dst_v.at[j]], ssem[b], add=True)

    def scatter_wait(j, b):
        pltpu.make_async_copy(rows[b], acc.at[dst_v.at[j]], ssem[b]).wait()

    @pl.loop(0, PH)
    def _(ph):
        pltpu.sync_copy(sv.at[pl.ds(ph * PCH, PCH)], src_v)
        pltpu.sync_copy(dv.at[pl.ds(ph * PCH, PCH)], dst_v)

        for b in range(NB):
            gather(b, b)

        @pl.loop(0, PCH // NB - 1)
        def _(jj):
            j = jj * NB
            for b in range(NB):
                gather_wait(j + b, b)
                scatter(j + b, b)
                scatter_wait(j + b, b)
                gather(j + b + NB, b)

        jl = PCH - NB
        for b in range(NB):
            gather_wait(jl + b, b)
            scatter(jl + b, b)
        for b in range(NB):
            scatter_wait(jl + b, b)

    plsc.subcore_barrier()
    pltpu.sync_copy(acc.at[pl.ds(sid * RPT, RPT)],
                    out_hbm.at[cid].at[pl.ds(sid * RPT, RPT)])


@functools.cache
def _sc_aggregate():
    mesh = plsc.VectorSubcoreMesh(core_axis_name="c", subcore_axis_name="s",
                                  num_cores=NC, num_subcores=NS)
    return pl.kernel(
        _sc_aggregate_body,
        out_type=jax.ShapeDtypeStruct((NC, NP, F), jnp.float32),
        mesh=mesh,
        scratch_types=[
            pltpu.VMEM((PCH, C), jnp.int32),
            pltpu.VMEM((PCH, C), jnp.int32),
            pltpu.VMEM_SHARED((NPA, F), jnp.float32),
            [pltpu.VMEM((C, F), jnp.float32) for _ in range(NB)],
            [pltpu.SemaphoreType.DMA for _ in range(NB)],
            [pltpu.SemaphoreType.DMA for _ in range(NB)],
        ],
        compiler_params=pltpu.CompilerParams(use_tc_tiling_on_sc=False),
    )


def _row_mask():
    rows = lax.broadcasted_iota(jnp.int32, (NP, 1), 0)
    return (rows < N).astype(jnp.float32)


def _mlp_block(t, W1, b1, g, be, W2, b2):
    rmask = _row_mask()
    y = jnp.dot(t, W1, preferred_element_type=jnp.float32) + b1
    mu = jnp.sum(y * rmask, axis=0, keepdims=True) * (1.0 / N)
    d = y - mu
    var = jnp.sum(d * d * rmask, axis=0, keepdims=True) * (1.0 / N)
    z = g * d * lax.rsqrt(var + EPS) + be
    z = jnp.maximum(z, 0.0)
    h2 = jnp.dot(z, W2, preferred_element_type=jnp.float32) + b2
    return jnp.maximum(h2, 0.0) * rmask


def _mlp_body(p_ref, h_ref, W1_ref, b1_ref, g_ref, be_ref, W2_ref, b2_ref,
              o_ref):
    t = p_ref[0] + p_ref[1] - h_ref[...]
    o_ref[...] = _mlp_block(t, W1_ref[...], b1_ref[...], g_ref[...],
                            be_ref[...], W2_ref[...], b2_ref[...])


_tc_mlp = pl.pallas_call(
    _mlp_body,
    out_shape=jax.ShapeDtypeStruct((NP, F), jnp.float32),
)


def _final_body(p_ref, h_ref, batch_ref, W1_ref, b1_ref, g_ref, be_ref,
                W2_ref, b2_ref, Wm1_ref, bm1_ref, Wm2_ref, bm2_ref, o_ref):
    t = p_ref[0] + p_ref[1] - h_ref[...]
    hh = _mlp_block(t, W1_ref[...], b1_ref[...], g_ref[...], be_ref[...],
                    W2_ref[...], b2_ref[...])
    cols = lax.broadcasted_iota(jnp.int32, (B, NP), 0)
    mask = (cols == batch_ref[...]).astype(jnp.float32)
    pooled = jnp.dot(mask, hh, preferred_element_type=jnp.float32)
    q = jnp.dot(pooled, Wm1_ref[...], preferred_element_type=jnp.float32) + bm1_ref[...]
    q = jnp.maximum(q, 0.0)
    o_ref[...] = jnp.dot(q, Wm2_ref[...], preferred_element_type=jnp.float32) + bm2_ref[...]


_tc_final = pl.pallas_call(
    _final_body,
    out_shape=jax.ShapeDtypeStruct((B, OUT), jnp.float32),
)


def kernel(x, edge_index, batch, batch_size,
           W1_0, b1_0, g_0, be_0, W2_0, b2_0,
           W1_1, b1_1, g_1, be_1, W2_1, b2_1,
           W1_2, b1_2, g_2, be_2, W2_2, b2_2,
           Wm1, bm1, Wm2, bm2):
    src = edge_index[0].reshape(NC, NS, CH, C)
    dst = edge_index[1].reshape(NC, NS, CH, C)
    batch2d = jnp.pad(batch, (0, NP - N), constant_values=B).reshape(1, NP)
    h = jnp.pad(x, ((0, NP - N), (0, 0)))

    def row(v):
        return v.reshape(1, -1)

    layers = [
        (W1_0, b1_0, g_0, be_0, W2_0, b2_0),
        (W1_1, b1_1, g_1, be_1, W2_1, b2_1),
    ]
    agg = _sc_aggregate()
    for (W1, b1, g, be, W2, b2) in layers:
        p = agg(h, src, dst)
        h = _tc_mlp(p, h, W1, row(b1), row(g), row(be), W2, row(b2))
    p = agg(h, src, dst)
    return _tc_final(p, h, batch2d, W1_2, row(b1_2), row(g_2), row(be_2),
                     W2_2, row(b2_2), Wm1, row(bm1), Wm2, row(bm2))
